# baseline (speedup 1.0000x reference)
import functools

import jax
import jax.numpy as jnp
from jax import lax
from jax.experimental import pallas as pl
from jax.experimental.pallas import tpu as pltpu
from jax.experimental.pallas import tpu_sc as plsc

NUM_CORES = 2
NUM_SUBCORES = 16
LANES = 16
NW = NUM_CORES * NUM_SUBCORES
BATCH = 16384
DIM = 16
CHUNK = 128
B_PER_W = BATCH // NW
NCHUNK = B_PER_W // CHUNK
EPS = 1e-7


def _sc_gamma_body(uids_hbm, iids_hbm, utab_hbm, itab_hbm, out_hbm,
                   uidx_v, iidx_v, urows, vrows, gout, sem_u, sem_v):
    wid = lax.axis_index("s") * NUM_CORES + lax.axis_index("c")
    base = wid * B_PER_W
    lane_iota = lax.iota(jnp.int32, LANES)
    lane_mask = [lane_iota == rl for rl in range(LANES)]
    for jc in range(NCHUNK):
        off = base + jc * CHUNK
        pltpu.sync_copy(uids_hbm.at[pl.ds(off, CHUNK)], uidx_v)
        pltpu.sync_copy(iids_hbm.at[pl.ds(off, CHUNK)], iidx_v)
        cu = pltpu.async_copy(utab_hbm.at[uidx_v], urows, sem_u)
        cv = pltpu.async_copy(itab_hbm.at[iidx_v], vrows, sem_v)
        cu.wait()
        cv.wait()
        @pl.loop(0, CHUNK // LANES)
        def _group(g):
            base_row = g * LANES
            acc_uv = jnp.zeros((LANES,), jnp.float32)
            acc_u = jnp.zeros((LANES,), jnp.float32)
            acc_v = jnp.zeros((LANES,), jnp.float32)
            for rl in range(LANES):
                u = urows[base_row + rl, :]
                v = vrows[base_row + rl, :]
                diff = u - v
                m = lane_mask[rl]
                acc_uv = jnp.where(m, jnp.sum(diff * diff), acc_uv)
                acc_u = jnp.where(m, jnp.sum(u * u), acc_u)
                acc_v = jnp.where(m, jnp.sum(v * v), acc_v)
            denom = jnp.maximum((1.0 - acc_u) * (1.0 - acc_v), EPS)
            gamma = jnp.maximum(1.0 + 2.0 * acc_uv / denom, 1.0 + EPS)
            gout[pl.ds(g * LANES, LANES)] = gamma

        pltpu.sync_copy(gout, out_hbm.at[pl.ds(off, CHUNK)])


@functools.partial(jax.jit, static_argnames=())
def _sc_gamma(user_ids, item_ids, user_weight, item_weight):
    mesh = plsc.VectorSubcoreMesh(core_axis_name="c", subcore_axis_name="s")
    return pl.kernel(
        _sc_gamma_body,
        out_type=jax.ShapeDtypeStruct((BATCH,), jnp.float32),
        mesh=mesh,
        scratch_types=[
            pltpu.VMEM((CHUNK,), jnp.int32),
            pltpu.VMEM((CHUNK,), jnp.int32),
            pltpu.VMEM((CHUNK, DIM), jnp.float32),
            pltpu.VMEM((CHUNK, DIM), jnp.float32),
            pltpu.VMEM((CHUNK,), jnp.float32),
            pltpu.SemaphoreType.DMA,
            pltpu.SemaphoreType.DMA,
        ],
        compiler_params=pltpu.CompilerParams(
            needs_layout_passes=False, use_tc_tiling_on_sc=False),
    )(user_ids, item_ids, user_weight, item_weight)


def _score_body(beta_ref, c_ref, g_ref, o_ref):
    g = g_ref[...]
    d = jnp.log(g + jnp.sqrt((g - 1.0) * (g + 1.0)))
    o_ref[...] = beta_ref[0] * d + c_ref[0]


def _tc_score(gamma, beta, c):
    g2 = gamma.reshape(BATCH // 128, 128)
    out = pl.pallas_call(
        _score_body,
        out_shape=jax.ShapeDtypeStruct(g2.shape, jnp.float32),
        in_specs=[
            pl.BlockSpec(memory_space=pltpu.SMEM),
            pl.BlockSpec(memory_space=pltpu.SMEM),
            pl.BlockSpec(memory_space=pltpu.VMEM),
        ],
        out_specs=pl.BlockSpec(memory_space=pltpu.VMEM),
    )(beta, c, g2)
    return out.reshape(-1)


def kernel(user_ids, item_ids, user_weight, item_weight, beta, c):
    gamma = _sc_gamma(user_ids.astype(jnp.int32), item_ids.astype(jnp.int32),
                      user_weight, item_weight)
    return _tc_score(gamma, beta, c)


# speedup vs baseline: 7.0813x; 7.0813x over previous
import jax
import jax.numpy as jnp
from jax import lax
from jax.experimental import pallas as pl
from jax.experimental.pallas import tpu as pltpu
from jax.experimental.pallas import tpu_sc as plsc

NUM_CORES = 2
NUM_SUBCORES = 16
LANES = 16
NW = NUM_CORES * NUM_SUBCORES
BATCH = 16384
DIM = 16
B_PER_W = BATCH // NW
NGROUP = B_PER_W // LANES
EPS = 1e-7


def _fetch_group(tab_hbm, ids_v, ring, sem, g, b):
    ids = ids_v[pl.ds(g * LANES, LANES)]
    tcs = ids >> 7
    for j in range(LANES):
        pltpu.async_copy(
            tab_hbm.at[:, pl.ds(tcs[j] * 128, 128)], ring.at[b, j], sem)


def _drain_group(tab_hbm, ring, sem, b):
    for j in range(LANES):
        pltpu.make_async_copy(
            tab_hbm.at[:, pl.ds(0, 128)], ring.at[b, j], sem).wait()


def _extract_group(ids_v, ring, cols, g, b):
    lanes = ids_v[pl.ds(g * LANES, LANES)] & 127
    bvec = jnp.full((LANES,), 0, jnp.int32) + b
    jvec = lax.iota(jnp.int32, LANES)
    for d in range(DIM):
        dvec = jnp.full((LANES,), d, jnp.int32)
        col = plsc.load_gather(ring, [bvec, jvec, dvec, lanes])
        cols[pl.ds(d * B_PER_W + g * LANES, LANES)] = col


def _gather_pass(tab_hbm, ids_v, ring, cols, sem):
    _fetch_group(tab_hbm, ids_v, ring, sem, 0, 0)

    @pl.loop(0, NGROUP - 1)
    def _grp(g):
        b = g % 2
        _fetch_group(tab_hbm, ids_v, ring, sem, g + 1, 1 - b)
        _drain_group(tab_hbm, ring, sem, b)
        _extract_group(ids_v, ring, cols, g, b)

    b_last = (NGROUP - 1) % 2
    _drain_group(tab_hbm, ring, sem, b_last)
    _extract_group(ids_v, ring, cols, NGROUP - 1, b_last)


def _sc_gamma_body(uids_hbm, iids_hbm, ut_hbm, vt_hbm, out_hbm,
                   uids_v, iids_v, ring, ucols, vcols, gout, sem):
    wid = lax.axis_index("s") * NUM_CORES + lax.axis_index("c")
    base = wid * B_PER_W
    pltpu.sync_copy(uids_hbm.at[pl.ds(base, B_PER_W)], uids_v)
    pltpu.sync_copy(iids_hbm.at[pl.ds(base, B_PER_W)], iids_v)

    _gather_pass(ut_hbm, uids_v, ring, ucols, sem)
    _gather_pass(vt_hbm, iids_v, ring, vcols, sem)

    @pl.loop(0, NGROUP)
    def _compute(g):
        s = g * LANES
        acc_uv = jnp.zeros((LANES,), jnp.float32)
        acc_u = jnp.zeros((LANES,), jnp.float32)
        acc_v = jnp.zeros((LANES,), jnp.float32)
        for d in range(DIM):
            ucol = ucols[pl.ds(d * B_PER_W + s, LANES)]
            vcol = vcols[pl.ds(d * B_PER_W + s, LANES)]
            diff = ucol - vcol
            acc_uv = acc_uv + diff * diff
            acc_u = acc_u + ucol * ucol
            acc_v = acc_v + vcol * vcol
        denom = jnp.maximum((1.0 - acc_u) * (1.0 - acc_v), EPS)
        gamma = jnp.maximum(1.0 + 2.0 * acc_uv / denom, 1.0 + EPS)
        gout[pl.ds(s, LANES)] = gamma

    pltpu.sync_copy(gout, out_hbm.at[pl.ds(base, B_PER_W)])


def _sc_gamma(user_ids, item_ids, ut, vt):
    mesh = plsc.VectorSubcoreMesh(core_axis_name="c", subcore_axis_name="s")
    return pl.kernel(
        _sc_gamma_body,
        out_type=jax.ShapeDtypeStruct((BATCH,), jnp.float32),
        mesh=mesh,
        scratch_types=[
            pltpu.VMEM((B_PER_W,), jnp.int32),
            pltpu.VMEM((B_PER_W,), jnp.int32),
            pltpu.VMEM((2, LANES, DIM, 128), jnp.float32),
            pltpu.VMEM((DIM * B_PER_W,), jnp.float32),
            pltpu.VMEM((DIM * B_PER_W,), jnp.float32),
            pltpu.VMEM((B_PER_W,), jnp.float32),
            pltpu.SemaphoreType.DMA,
        ],
        compiler_params=pltpu.CompilerParams(
            needs_layout_passes=False, use_tc_tiling_on_sc=True),
    )(user_ids, item_ids, ut, vt)


def _score_body(beta_ref, c_ref, g_ref, o_ref):
    g = g_ref[...]
    d = jnp.log(g + jnp.sqrt((g - 1.0) * (g + 1.0)))
    o_ref[...] = beta_ref[0] * d + c_ref[0]


def _tc_score(gamma, beta, c):
    g2 = gamma.reshape(BATCH // 128, 128)
    out = pl.pallas_call(
        _score_body,
        out_shape=jax.ShapeDtypeStruct(g2.shape, jnp.float32),
        in_specs=[
            pl.BlockSpec(memory_space=pltpu.SMEM),
            pl.BlockSpec(memory_space=pltpu.SMEM),
            pl.BlockSpec(memory_space=pltpu.VMEM),
        ],
        out_specs=pl.BlockSpec(memory_space=pltpu.VMEM),
    )(beta, c, g2)
    return out.reshape(-1)


def kernel(user_ids, item_ids, user_weight, item_weight, beta, c):
    gamma = _sc_gamma(user_ids.astype(jnp.int32), item_ids.astype(jnp.int32),
                      user_weight.T, item_weight.T)
    return _tc_score(gamma, beta, c)
